# speedup vs baseline: 11.1887x; 11.1887x over previous
import functools

import jax
import jax.numpy as jnp
from jax import lax
from jax.experimental import pallas as pl
from jax.experimental.pallas import tpu as pltpu
from jax.experimental.pallas import tpu_sc as plsc

B = 4096
L = 200
D = 128
LA = 128
LB = L - LA
LANES = 16
NCHUNK = D // LANES

_info = plsc.get_sparse_core_info()
_NC = _info.num_cores
_NS = _info.num_subcores
NW = _NC * _NS
RPW = B // NW


@functools.partial(
    pl.kernel,
    out_type=jax.ShapeDtypeStruct((B, D), jnp.float32),
    mesh=plsc.VectorSubcoreMesh(core_axis_name="c", subcore_axis_name="s"),
    scratch_types=[
        pltpu.VMEM((RPW, LA), jnp.int32),
        pltpu.VMEM((RPW, LB), jnp.int32),
        pltpu.VMEM((2, L, D), jnp.float32),
        pltpu.VMEM((RPW, D), jnp.float32),
        pltpu.SemaphoreType.DMA,
        pltpu.SemaphoreType.DMA,
    ],
)
def _bow_max_kernel(idx_a_hbm, idx_b_hbm, table_hbm, out_hbm,
                    idx_a_v, idx_b_v, rows_v, out_v, sem0, sem1):
    wid = lax.axis_index("s") * _NC + lax.axis_index("c")
    base = wid * RPW

    pltpu.sync_copy(idx_a_hbm.at[pl.ds(base, RPW), :], idx_a_v)
    pltpu.sync_copy(idx_b_hbm.at[pl.ds(base, RPW), :], idx_b_v)

    sems = (sem0, sem1)

    def gather(r, slot):
        sem = sems[slot]
        a = pltpu.make_async_copy(
            table_hbm.at[idx_a_v.at[r]], rows_v.at[slot, pl.ds(0, LA)], sem)
        b = pltpu.make_async_copy(
            table_hbm.at[idx_b_v.at[r]], rows_v.at[slot, pl.ds(LA, LB)], sem)
        return a, b

    def start_gather(r, slot):
        a, b = gather(r, slot)
        a.start()
        b.start()

    def wait_gather(r, slot):
        a, b = gather(r, slot)
        a.wait()
        b.wait()

    start_gather(0, 0)

    def do_row(r, slot):
        wait_gather(r, slot)

        @pl.when(r < RPW - 1)
        def _():
            start_gather(r + 1, 1 - slot)

        def body(j, accs):
            return tuple(
                jnp.maximum(accs[c], rows_v[slot, j, pl.ds(c * LANES, LANES)])
                for c in range(NCHUNK))

        init = tuple(
            rows_v[slot, 0, pl.ds(c * LANES, LANES)] for c in range(NCHUNK))
        accs = lax.fori_loop(1, L, body, init)
        for c in range(NCHUNK):
            out_v[r, pl.ds(c * LANES, LANES)] = accs[c]

    def outer(g, _):
        do_row(2 * g, 0)
        do_row(2 * g + 1, 1)
        return 0

    lax.fori_loop(0, RPW // 2, outer, 0)

    pltpu.sync_copy(out_v, out_hbm.at[pl.ds(base, RPW), :])


def kernel(inputs, emb_weight):
    idx_a = inputs[:, :LA]
    idx_b = inputs[:, LA:]
    return _bow_max_kernel(idx_a, idx_b, emb_weight)


# speedup vs baseline: 17.0075x; 1.5201x over previous
import functools

import jax
import jax.numpy as jnp
from jax import lax
from jax.experimental import pallas as pl
from jax.experimental.pallas import tpu as pltpu
from jax.experimental.pallas import tpu_sc as plsc

B = 4096
L = 200
D = 128
LA = 128
LB = L - LA
LANES = 16
NCHUNK = D // LANES

_info = plsc.get_sparse_core_info()
_NC = _info.num_cores
_NS = _info.num_subcores
NW = _NC * _NS
RPW = B // NW


@functools.partial(
    pl.kernel,
    out_type=jax.ShapeDtypeStruct((B, D), jnp.float32),
    mesh=plsc.VectorSubcoreMesh(core_axis_name="c", subcore_axis_name="s"),
    scratch_types=[
        pltpu.VMEM((RPW, LA), jnp.int32),
        pltpu.VMEM((RPW, LB), jnp.int32),
        pltpu.VMEM((3, L, D), jnp.float32),
        pltpu.VMEM((RPW, D), jnp.float32),
        pltpu.SemaphoreType.DMA,
        pltpu.SemaphoreType.DMA,
        pltpu.SemaphoreType.DMA,
    ],
)
def _bow_max_kernel(idx_a_hbm, idx_b_hbm, table_hbm, out_hbm,
                    idx_a_v, idx_b_v, rows_v, out_v, sem0, sem1, sem2):
    wid = lax.axis_index("s") * _NC + lax.axis_index("c")
    base = wid * RPW

    pltpu.sync_copy(idx_a_hbm.at[pl.ds(base, RPW), :], idx_a_v)
    pltpu.sync_copy(idx_b_hbm.at[pl.ds(base, RPW), :], idx_b_v)

    sems = (sem0, sem1, sem2)
    NBUF = 3

    def gather(r, slot):
        sem = sems[slot]
        a = pltpu.make_async_copy(
            table_hbm.at[idx_a_v.at[r]], rows_v.at[slot, pl.ds(0, LA)], sem)
        b = pltpu.make_async_copy(
            table_hbm.at[idx_b_v.at[r]], rows_v.at[slot, pl.ds(LA, LB)], sem)
        return a, b

    def start_gather(r, slot):
        a, b = gather(r, slot)
        a.start()
        b.start()

    def wait_gather(r, slot):
        a, b = gather(r, slot)
        a.wait()
        b.wait()

    for p in range(NBUF - 1):
        start_gather(p, p)

    def do_row(r, slot):
        wait_gather(r, slot)

        @pl.when(r < RPW - (NBUF - 1))
        def _():
            start_gather(r + NBUF - 1, (slot + NBUF - 1) % NBUF)

        def body(j, accs):
            return tuple(
                jnp.maximum(accs[c], rows_v[slot, j, pl.ds(c * LANES, LANES)])
                for c in range(NCHUNK))

        init = tuple(
            rows_v[slot, 0, pl.ds(c * LANES, LANES)] for c in range(NCHUNK))
        accs = lax.fori_loop(1, L, body, init)
        for c in range(NCHUNK):
            out_v[r, pl.ds(c * LANES, LANES)] = accs[c]

    def outer(g, _):
        for b in range(NBUF):
            do_row(NBUF * g + b, b)
        return 0

    assert RPW % NBUF == 0 or True
    n_full = RPW // NBUF
    lax.fori_loop(0, n_full, outer, 0)
    for b in range(RPW - n_full * NBUF):
        do_row(n_full * NBUF + b, b)

    pltpu.sync_copy(out_v, out_hbm.at[pl.ds(base, RPW), :])


def kernel(inputs, emb_weight):
    idx_a = inputs[:, :LA]
    idx_b = inputs[:, LA:]
    return _bow_max_kernel(idx_a, idx_b, emb_weight)
